# baseline (speedup 1.0000x reference)
import jax
import jax.numpy as jnp
from jax import lax
from jax.experimental import pallas as pl
from jax.experimental.pallas import tpu as pltpu
from jax.experimental.pallas import tpu_sc as plsc

NC = 2
NS = 16
NW = NC * NS
LANES = 16

D = 64
HIST = 50
NUM_NEG = 4
BATCH = 4096
TOTAL_ROWS = (2 + NUM_NEG) * BATCH
ROWS_PER_TILE = TOTAL_ROWS // NW
IDS_PER_TILE = ROWS_PER_TILE * HIST
G = 8
IDS_PER_G = G * HIST
NGROUPS = ROWS_PER_TILE // G
NBUF = 3
SPLITS = [(0, 128), (128, 128), (256, 128), (384, 16)]


def _lane_sum(x):
    lane = lax.iota(jnp.int32, LANES)
    for k in (1, 2, 4, 8):
        x = x + jnp.take_along_axis(x, lane ^ k, axis=0)
    return x


def _fast_rsqrt(sv):
    iv = lax.bitcast_convert_type(sv, jnp.int32)
    iv = jnp.int32(0x5F3759DF) - (iv >> 1)
    y = lax.bitcast_convert_type(iv, jnp.float32)
    half = sv * 0.5
    for _ in range(3):
        y = y * (1.5 - half * y * y)
    return y


def _sc_body(ids_hbm, table_hbm, out_hbm,
             ids_v, rows_a, rows_b, rows_c, out_a, out_b, out_c,
             sem_a, sem_b, sem_c, out_sem):
    wid = lax.axis_index("s") * NC + lax.axis_index("c")
    row0 = wid * ROWS_PER_TILE
    id_base = row0 * HIST

    rows_bufs = (rows_a, rows_b, rows_c)
    out_bufs = (out_a, out_b, out_c)
    sems = (sem_a, sem_b, sem_c)

    pltpu.sync_copy(ids_hbm.at[pl.ds(id_base, IDS_PER_TILE)], ids_v)

    def fetch(g, b):
        @pl.when(g < NGROUPS)
        def _():
            for off, n in SPLITS:
                pltpu.async_copy(
                    table_hbm.at[ids_v.at[pl.ds(g * IDS_PER_G + off, n)]],
                    rows_bufs[b].at[pl.ds(off, n)], sems[b])

    def wait_gathers(g, b):
        for off, n in SPLITS:
            pltpu.make_async_copy(
                table_hbm.at[ids_v.at[pl.ds(g * IDS_PER_G + off, n)]],
                rows_bufs[b].at[pl.ds(off, n)], sems[b]).wait()

    def accum_group(g, b):
        rows = rows_bufs[b]
        ov = out_bufs[b]
        @pl.when(g >= NBUF)
        def _():
            pltpu.make_async_copy(
                ov, out_hbm.at[pl.ds(row0 + (g - NBUF) * G, G)], out_sem).wait()
        for r in range(G):
            base = r * HIST

            def lbody(l, acc):
                a0, a1, a2, a3 = acc
                i = base + 2 * l
                a0 = a0 + rows[i, pl.ds(0, 16)] + rows[i + 1, pl.ds(0, 16)]
                a1 = a1 + rows[i, pl.ds(16, 16)] + rows[i + 1, pl.ds(16, 16)]
                a2 = a2 + rows[i, pl.ds(32, 16)] + rows[i + 1, pl.ds(32, 16)]
                a3 = a3 + rows[i, pl.ds(48, 16)] + rows[i + 1, pl.ds(48, 16)]
                return (a0, a1, a2, a3)

            z = jnp.zeros((LANES,), jnp.float32)
            a0, a1, a2, a3 = lax.fori_loop(0, HIST // 2, lbody, (z, z, z, z),
                                           unroll=5)
            ss = a0 * a0 + a1 * a1 + a2 * a2 + a3 * a3
            inv = _fast_rsqrt(_lane_sum(ss))
            ov[r, pl.ds(0, 16)] = a0 * inv
            ov[r, pl.ds(16, 16)] = a1 * inv
            ov[r, pl.ds(32, 16)] = a2 * inv
            ov[r, pl.ds(48, 16)] = a3 * inv
        pltpu.async_copy(ov, out_hbm.at[pl.ds(row0 + g * G, G)], out_sem)

    for b in range(NBUF):
        fetch(b, b)

    def step(i, carry):
        for b in range(NBUF):
            g = i * NBUF + b
            wait_gathers(g, b)
            accum_group(g, b)
            fetch(g + NBUF, b)
        return carry

    lax.fori_loop(0, NGROUPS // NBUF, step, 0)

    for b in range(NBUF):
        g = NGROUPS - NBUF + b
        pltpu.make_async_copy(
            out_bufs[b], out_hbm.at[pl.ds(row0 + g * G, G)], out_sem).wait()


@jax.jit
def _run(ids_flat, table):
    mesh = plsc.VectorSubcoreMesh(core_axis_name="c", subcore_axis_name="s",
                                  num_cores=NC, num_subcores=NS)
    return pl.kernel(
        _sc_body,
        out_type=jax.ShapeDtypeStruct((TOTAL_ROWS, D), jnp.float32),
        mesh=mesh,
        compiler_params=pltpu.CompilerParams(use_tc_tiling_on_sc=False),
        scratch_types=[
            pltpu.VMEM((IDS_PER_TILE,), jnp.int32),
            pltpu.VMEM((IDS_PER_G, D), jnp.float32),
            pltpu.VMEM((IDS_PER_G, D), jnp.float32),
            pltpu.VMEM((IDS_PER_G, D), jnp.float32),
            pltpu.VMEM((G, D), jnp.float32),
            pltpu.VMEM((G, D), jnp.float32),
            pltpu.VMEM((G, D), jnp.float32),
            pltpu.SemaphoreType.DMA,
            pltpu.SemaphoreType.DMA,
            pltpu.SemaphoreType.DMA,
            pltpu.SemaphoreType.DMA,
        ],
    )(ids_flat, table)


def kernel(anchor_input_ids, positive_input_ids, negative_input_ids, embedding_table):
    ids_flat = jnp.concatenate([
        anchor_input_ids.reshape(-1),
        positive_input_ids.reshape(-1),
        negative_input_ids.reshape(-1),
    ])
    out = _run(ids_flat, embedding_table)
    anchor = out[:BATCH]
    positive = out[BATCH:2 * BATCH]
    negative = out[2 * BATCH:].reshape(NUM_NEG, BATCH, D)
    return (anchor, positive, negative)


# speedup vs baseline: 1.0769x; 1.0769x over previous
import jax
import jax.numpy as jnp
from jax import lax
from jax.experimental import pallas as pl
from jax.experimental.pallas import tpu as pltpu
from jax.experimental.pallas import tpu_sc as plsc

NC = 2
NS = 16
NW = NC * NS
LANES = 16

D = 64
HIST = 50
NUM_NEG = 4
BATCH = 4096
TOTAL_ROWS = (2 + NUM_NEG) * BATCH
ROWS_PER_TILE = TOTAL_ROWS // NW
IDS_PER_TILE = ROWS_PER_TILE * HIST
G = 8
IDS_PER_G = G * HIST
NGROUPS = ROWS_PER_TILE // G
NBUF = 3
SPLITS = [(0, 128), (128, 128), (256, 128), (384, 16)]


def _lane_sum(x):
    lane = lax.iota(jnp.int32, LANES)
    for k in (1, 2, 4, 8):
        x = x + jnp.take_along_axis(x, lane ^ k, axis=0)
    return x


def _fast_rsqrt(sv):
    iv = lax.bitcast_convert_type(sv, jnp.int32)
    iv = jnp.int32(0x5F3759DF) - (iv >> 1)
    y = lax.bitcast_convert_type(iv, jnp.float32)
    half = sv * 0.5
    for _ in range(3):
        y = y * (1.5 - half * y * y)
    return y


def _sc_body(ids_hbm, table_hbm, out_hbm,
             ids_v, rows_a, rows_b, rows_c, out_a, out_b, out_c,
             sem_a, sem_b, sem_c, out_sem):
    wid = lax.axis_index("s") * NC + lax.axis_index("c")
    row0 = wid * ROWS_PER_TILE
    id_base = row0 * HIST

    rows_bufs = (rows_a, rows_b, rows_c)
    out_bufs = (out_a, out_b, out_c)
    sems = (sem_a, sem_b, sem_c)

    pltpu.sync_copy(ids_hbm.at[pl.ds(id_base, IDS_PER_TILE)], ids_v)

    def fetch(g, b):
        @pl.when(g < NGROUPS)
        def _():
            for off, n in SPLITS:
                pltpu.async_copy(
                    table_hbm.at[ids_v.at[pl.ds(g * IDS_PER_G + off, n)]],
                    rows_bufs[b].at[pl.ds(off, n)], sems[b])

    def wait_gathers(g, b):
        for off, n in SPLITS:
            pltpu.make_async_copy(
                table_hbm.at[ids_v.at[pl.ds(g * IDS_PER_G + off, n)]],
                rows_bufs[b].at[pl.ds(off, n)], sems[b]).wait()

    def accum_group(g, b):
        rows = rows_bufs[b]
        ov = out_bufs[b]
        @pl.when(g >= NBUF)
        def _():
            pltpu.make_async_copy(
                ov, out_hbm.at[pl.ds(row0 + (g - NBUF) * G, G)], out_sem).wait()
        for r in range(G):
            base = r * HIST

            def lbody(l, acc):
                a0, a1, a2, a3 = acc
                i = base + 2 * l
                a0 = a0 + rows[i, pl.ds(0, 16)] + rows[i + 1, pl.ds(0, 16)]
                a1 = a1 + rows[i, pl.ds(16, 16)] + rows[i + 1, pl.ds(16, 16)]
                a2 = a2 + rows[i, pl.ds(32, 16)] + rows[i + 1, pl.ds(32, 16)]
                a3 = a3 + rows[i, pl.ds(48, 16)] + rows[i + 1, pl.ds(48, 16)]
                return (a0, a1, a2, a3)

            z = jnp.zeros((LANES,), jnp.float32)
            a0, a1, a2, a3 = lax.fori_loop(0, HIST // 2, lbody, (z, z, z, z))
            ss = a0 * a0 + a1 * a1 + a2 * a2 + a3 * a3
            inv = _fast_rsqrt(_lane_sum(ss))
            ov[r, pl.ds(0, 16)] = a0 * inv
            ov[r, pl.ds(16, 16)] = a1 * inv
            ov[r, pl.ds(32, 16)] = a2 * inv
            ov[r, pl.ds(48, 16)] = a3 * inv
        pltpu.async_copy(ov, out_hbm.at[pl.ds(row0 + g * G, G)], out_sem)

    for b in range(NBUF):
        fetch(b, b)

    def step(i, carry):
        for b in range(NBUF):
            g = i * NBUF + b
            wait_gathers(g, b)
            accum_group(g, b)
            fetch(g + NBUF, b)
        return carry

    lax.fori_loop(0, NGROUPS // NBUF, step, 0)

    for b in range(NBUF):
        g = NGROUPS - NBUF + b
        pltpu.make_async_copy(
            out_bufs[b], out_hbm.at[pl.ds(row0 + g * G, G)], out_sem).wait()


@jax.jit
def _run(ids_flat, table):
    mesh = plsc.VectorSubcoreMesh(core_axis_name="c", subcore_axis_name="s",
                                  num_cores=NC, num_subcores=NS)
    return pl.kernel(
        _sc_body,
        out_type=jax.ShapeDtypeStruct((TOTAL_ROWS, D), jnp.float32),
        mesh=mesh,
        compiler_params=pltpu.CompilerParams(use_tc_tiling_on_sc=False),
        scratch_types=[
            pltpu.VMEM((IDS_PER_TILE,), jnp.int32),
            pltpu.VMEM((IDS_PER_G, D), jnp.float32),
            pltpu.VMEM((IDS_PER_G, D), jnp.float32),
            pltpu.VMEM((IDS_PER_G, D), jnp.float32),
            pltpu.VMEM((G, D), jnp.float32),
            pltpu.VMEM((G, D), jnp.float32),
            pltpu.VMEM((G, D), jnp.float32),
            pltpu.SemaphoreType.DMA,
            pltpu.SemaphoreType.DMA,
            pltpu.SemaphoreType.DMA,
            pltpu.SemaphoreType.DMA,
        ],
    )(ids_flat, table)


def kernel(anchor_input_ids, positive_input_ids, negative_input_ids, embedding_table):
    ids_flat = jnp.concatenate([
        anchor_input_ids.reshape(-1),
        positive_input_ids.reshape(-1),
        negative_input_ids.reshape(-1),
    ])
    out = _run(ids_flat, embedding_table)
    anchor = out[:BATCH]
    positive = out[BATCH:2 * BATCH]
    negative = out[2 * BATCH:].reshape(NUM_NEG, BATCH, D)
    return (anchor, positive, negative)
